# baseline (speedup 1.0000x reference)
import functools

import jax
import jax.numpy as jnp
from jax import lax
from jax.experimental import pallas as pl
from jax.experimental.pallas import tpu as pltpu
from jax.experimental.pallas import tpu_sc as plsc

_NUM_CORES = 2
_NUM_SUBCORES = 16
_NUM_WORKERS = _NUM_CORES * _NUM_SUBCORES
_CHUNK = 16
_TC_ROWS = 512


def _sc_body(batch, rows_per_worker, table_hbm, out_hbm,
             buf0, buf1, lsem0, lsem1, ssem0, ssem1):
    wid = lax.axis_index("s") * _NUM_CORES + lax.axis_index("c")
    base = wid * rows_per_worker
    bufs = (buf0, buf1)
    lsems = (lsem0, lsem1)
    ssems = (ssem0, ssem1)
    nch = rows_per_worker // _CHUNK

    loads = [None] * nch
    stores = [[] for _ in range(nch)]
    loads[0] = pltpu.async_copy(table_hbm.at[pl.ds(base, _CHUNK)], buf0, lsem0)
    for c in range(nch):
        pb = c % 2
        if c + 1 < nch:
            for d in stores[c - 1] if c >= 1 else ():
                d.wait()
            loads[c + 1] = pltpu.async_copy(
                table_hbm.at[pl.ds(base + (c + 1) * _CHUNK, _CHUNK)],
                bufs[(c + 1) % 2], lsems[(c + 1) % 2])
        loads[c].wait()
        r0 = base + c * _CHUNK
        for b in range(batch):
            stores[c].append(pltpu.async_copy(
                bufs[pb], out_hbm.at[b, pl.ds(r0, _CHUNK)], ssems[pb]))
    for c in (nch - 2, nch - 1):
        if c >= 0:
            for d in stores[c]:
                d.wait()


def _tc_body(tab_ref, acc_ref, out_ref):
    out_ref[...] = tab_ref[...][None, :, :]


def kernel(sequence, table):
    batch, seq_len = sequence.shape
    hidden = table.shape[1]
    rows_per_worker = seq_len // _NUM_WORKERS

    mesh = plsc.VectorSubcoreMesh(core_axis_name="c", subcore_axis_name="s")
    sc_kernel = pl.kernel(
        functools.partial(_sc_body, batch, rows_per_worker),
        out_type=jax.ShapeDtypeStruct((batch, seq_len, hidden), table.dtype),
        mesh=mesh,
        scratch_types=[
            pltpu.VMEM((_CHUNK, hidden), table.dtype),
            pltpu.VMEM((_CHUNK, hidden), table.dtype),
            pltpu.SemaphoreType.DMA,
            pltpu.SemaphoreType.DMA,
            pltpu.SemaphoreType.DMA,
            pltpu.SemaphoreType.DMA,
        ],
    )
    return sc_kernel(table)


# speedup vs baseline: 1.0797x; 1.0797x over previous
import functools

import jax
import jax.numpy as jnp
from jax import lax
from jax.experimental import pallas as pl
from jax.experimental.pallas import tpu as pltpu
from jax.experimental.pallas import tpu_sc as plsc

_NUM_CORES = 2
_NUM_SUBCORES = 16
_NUM_WORKERS = _NUM_CORES * _NUM_SUBCORES
_BUF_ROWS = 56


def _chunk_schedule(rows_per_worker):
    sizes = []
    left = rows_per_worker
    while left > _BUF_ROWS + 8:
        sizes.append(_BUF_ROWS)
        left -= _BUF_ROWS
    if left > 8:
        sizes.append(left - 8)
        left = 8
    sizes.append(left)
    return sizes
_TC_ROWS = 512


def _sc_body(batch, rows_per_worker, table_hbm, out_hbm,
             buf0, buf1, lsem0, lsem1, ssem0, ssem1):
    wid = lax.axis_index("s") * _NUM_CORES + lax.axis_index("c")
    base = wid * rows_per_worker
    bufs = (buf0, buf1)
    lsems = (lsem0, lsem1)
    ssems = (ssem0, ssem1)
    sizes = _chunk_schedule(rows_per_worker)
    offs = [0]
    for s in sizes:
        offs.append(offs[-1] + s)
    nch = len(sizes)

    loads = [None] * nch
    stores = [[] for _ in range(nch)]
    loads[0] = pltpu.async_copy(
        table_hbm.at[pl.ds(base, sizes[0])], buf0.at[pl.ds(0, sizes[0])],
        lsem0)
    for c in range(nch):
        pb = c % 2
        if c + 1 < nch:
            for d in stores[c - 1] if c >= 1 else ():
                d.wait()
            loads[c + 1] = pltpu.async_copy(
                table_hbm.at[pl.ds(base + offs[c + 1], sizes[c + 1])],
                bufs[(c + 1) % 2].at[pl.ds(0, sizes[c + 1])],
                lsems[(c + 1) % 2])
        loads[c].wait()
        r0 = base + offs[c]
        for b in range(batch):
            stores[c].append(pltpu.async_copy(
                bufs[pb].at[pl.ds(0, sizes[c])],
                out_hbm.at[b, pl.ds(r0, sizes[c])], ssems[pb]))
    for c in (nch - 2, nch - 1):
        if c >= 0:
            for d in stores[c]:
                d.wait()


def _tc_body(tab_ref, acc_ref, out_ref):
    out_ref[...] = tab_ref[...][None, :, :]


def kernel(sequence, table):
    batch, seq_len = sequence.shape
    hidden = table.shape[1]
    rows_per_worker = seq_len // _NUM_WORKERS

    mesh = plsc.VectorSubcoreMesh(core_axis_name="c", subcore_axis_name="s")
    sc_kernel = pl.kernel(
        functools.partial(_sc_body, batch, rows_per_worker),
        out_type=jax.ShapeDtypeStruct((batch, seq_len, hidden), table.dtype),
        mesh=mesh,
        scratch_types=[
            pltpu.VMEM((_BUF_ROWS, hidden), table.dtype),
            pltpu.VMEM((_BUF_ROWS, hidden), table.dtype),
            pltpu.SemaphoreType.DMA,
            pltpu.SemaphoreType.DMA,
            pltpu.SemaphoreType.DMA,
            pltpu.SemaphoreType.DMA,
        ],
    )
    return sc_kernel(table)


# speedup vs baseline: 1.0964x; 1.0155x over previous
import functools

import jax
import jax.numpy as jnp
from jax import lax
from jax.experimental import pallas as pl
from jax.experimental.pallas import tpu as pltpu
from jax.experimental.pallas import tpu_sc as plsc

_NUM_CORES = 2
_NUM_SUBCORES = 16
_NUM_WORKERS = _NUM_CORES * _NUM_SUBCORES
_BUF_ROWS = 56


def _chunk_schedule(rows_per_worker):
    sizes = []
    left = rows_per_worker
    while left > _BUF_ROWS:
        sizes.append(_BUF_ROWS)
        left -= _BUF_ROWS
    sizes.append(left)
    return sizes
_TC_ROWS = 512


def _sc_body(batch, rows_per_worker, table_hbm, out_hbm,
             buf0, buf1, lsem0, lsem1, ssem0, ssem1):
    wid = lax.axis_index("s") * _NUM_CORES + lax.axis_index("c")
    base = wid * rows_per_worker
    bufs = (buf0, buf1)
    lsems = (lsem0, lsem1)
    ssems = (ssem0, ssem1)
    sizes = _chunk_schedule(rows_per_worker)
    offs = [0]
    for s in sizes:
        offs.append(offs[-1] + s)
    nch = len(sizes)

    loads = [None] * nch
    stores = [[] for _ in range(nch)]
    loads[0] = pltpu.async_copy(
        table_hbm.at[pl.ds(base, sizes[0])], buf0.at[pl.ds(0, sizes[0])],
        lsem0)
    for c in range(nch):
        pb = c % 2
        if c + 1 < nch:
            for d in stores[c - 1] if c >= 1 else ():
                d.wait()
            loads[c + 1] = pltpu.async_copy(
                table_hbm.at[pl.ds(base + offs[c + 1], sizes[c + 1])],
                bufs[(c + 1) % 2].at[pl.ds(0, sizes[c + 1])],
                lsems[(c + 1) % 2])
        loads[c].wait()
        r0 = base + offs[c]
        for b in range(batch):
            stores[c].append(pltpu.async_copy(
                bufs[pb].at[pl.ds(0, sizes[c])],
                out_hbm.at[b, pl.ds(r0, sizes[c])], ssems[pb]))
    for c in (nch - 2, nch - 1):
        if c >= 0:
            for d in stores[c]:
                d.wait()


def _tc_body(tab_ref, acc_ref, out_ref):
    out_ref[...] = tab_ref[...][None, :, :]


def kernel(sequence, table):
    batch, seq_len = sequence.shape
    hidden = table.shape[1]
    rows_per_worker = seq_len // _NUM_WORKERS

    mesh = plsc.VectorSubcoreMesh(core_axis_name="c", subcore_axis_name="s")
    sc_kernel = pl.kernel(
        functools.partial(_sc_body, batch, rows_per_worker),
        out_type=jax.ShapeDtypeStruct((batch, seq_len, hidden), table.dtype),
        mesh=mesh,
        scratch_types=[
            pltpu.VMEM((_BUF_ROWS, hidden), table.dtype),
            pltpu.VMEM((_BUF_ROWS, hidden), table.dtype),
            pltpu.SemaphoreType.DMA,
            pltpu.SemaphoreType.DMA,
            pltpu.SemaphoreType.DMA,
            pltpu.SemaphoreType.DMA,
        ],
    )
    return sc_kernel(table)
